# baseline (speedup 1.0000x reference)
import functools

import jax
import jax.numpy as jnp
from jax import lax
from jax.experimental import pallas as pl
from jax.experimental.pallas import tpu as pltpu
from jax.experimental.pallas import tpu_sc as plsc

VOCAB = 100000
EMBED = 64
BATCH = 1024

_VT = 4096


def _make_sc_gather(V, D, B):
    info = plsc.get_sparse_core_info()
    NC, NS = info.num_cores, info.num_subcores
    NW = NC * NS
    b_per_w = B // NW
    mesh = plsc.VectorSubcoreMesh(core_axis_name="c", subcore_axis_name="s")

    @functools.partial(
        pl.kernel,
        mesh=mesh,
        compiler_params=pltpu.CompilerParams(use_tc_tiling_on_sc=False),
        out_type=jax.ShapeDtypeStruct((B, D), jnp.float32),
        scratch_types=[
            pltpu.VMEM((b_per_w,), jnp.int32),
            pltpu.VMEM((b_per_w, D), jnp.float32),
            pltpu.SemaphoreType.DMA,
        ],
    )
    def gather(table_hbm, idx_hbm, out_hbm, idx_v, rows_v, sem):
        wid = lax.axis_index("s") * NC + lax.axis_index("c")
        base = wid * b_per_w
        pltpu.sync_copy(idx_hbm.at[pl.ds(base, b_per_w)], idx_v)
        pltpu.async_copy(table_hbm.at[idx_v], rows_v, sem).wait()
        pltpu.sync_copy(rows_v, out_hbm.at[pl.ds(base, b_per_w)])

    return gather


def _matmul_body(h_ref, w_ref, o_ref):
    o_ref[...] = lax.dot_general(
        h_ref[...].astype(jnp.bfloat16), w_ref[...].astype(jnp.bfloat16),
        (((1,), (1,)), ((), ())),
        preferred_element_type=jnp.float32,
    )


def kernel(X, W_emb, WT_w):
    hidden = _make_sc_gather(VOCAB, EMBED, BATCH)(W_emb, X.astype(jnp.int32))
    n_tiles = pl.cdiv(VOCAB, _VT)
    out = pl.pallas_call(
        _matmul_body,
        grid=(n_tiles,),
        in_specs=[
            pl.BlockSpec((BATCH, EMBED), lambda i: (0, 0)),
            pl.BlockSpec((_VT, EMBED), lambda i: (i, 0)),
        ],
        out_specs=pl.BlockSpec((BATCH, _VT), lambda i: (0, i)),
        out_shape=jax.ShapeDtypeStruct((BATCH, VOCAB), jnp.float32),
    )(hidden, WT_w)
    return out


# speedup vs baseline: 1.0020x; 1.0020x over previous
import functools

import jax
import jax.numpy as jnp
from jax import lax
from jax.experimental import pallas as pl
from jax.experimental.pallas import tpu as pltpu
from jax.experimental.pallas import tpu_sc as plsc

VOCAB = 100000
EMBED = 64
BATCH = 1024

_VT = 4096


def _make_sc_gather(V, D, B):
    info = plsc.get_sparse_core_info()
    NC, NS = info.num_cores, info.num_subcores
    NW = NC * NS
    b_per_w = B // NW
    mesh = plsc.VectorSubcoreMesh(core_axis_name="c", subcore_axis_name="s")

    @functools.partial(
        pl.kernel,
        mesh=mesh,
        compiler_params=pltpu.CompilerParams(use_tc_tiling_on_sc=False),
        out_type=jax.ShapeDtypeStruct((B, D), jnp.float32),
        scratch_types=[
            pltpu.VMEM((b_per_w,), jnp.int32),
            pltpu.VMEM((b_per_w, D), jnp.float32),
            pltpu.SemaphoreType.DMA,
        ],
    )
    def gather(table_hbm, idx_hbm, out_hbm, idx_v, rows_v, sem):
        wid = lax.axis_index("s") * NC + lax.axis_index("c")
        base = wid * b_per_w
        pltpu.sync_copy(idx_hbm.at[pl.ds(base, b_per_w)], idx_v)
        pltpu.async_copy(table_hbm.at[idx_v], rows_v, sem).wait()
        pltpu.sync_copy(rows_v, out_hbm.at[pl.ds(base, b_per_w)])

    return gather


def _matmul_body(h_ref, w_ref, o_ref):
    o_ref[...] = lax.dot_general(
        h_ref[...].astype(jnp.bfloat16), w_ref[...].astype(jnp.bfloat16),
        (((1,), (1,)), ((), ())),
        preferred_element_type=jnp.float32,
    )


def kernel(X, W_emb, WT_w):
    hidden = _make_sc_gather(VOCAB, EMBED, BATCH)(W_emb, X.astype(jnp.int32))
    n_tiles = pl.cdiv(VOCAB, _VT)
    out = pl.pallas_call(
        _matmul_body,
        grid=(n_tiles,),
        compiler_params=pltpu.CompilerParams(
            dimension_semantics=("parallel",),
        ),
        in_specs=[
            pl.BlockSpec((BATCH, EMBED), lambda i: (0, 0)),
            pl.BlockSpec((_VT, EMBED), lambda i: (i, 0)),
        ],
        out_specs=pl.BlockSpec((BATCH, _VT), lambda i: (0, i)),
        out_shape=jax.ShapeDtypeStruct((BATCH, VOCAB), jnp.float32),
    )(hidden, WT_w)
    return out


# speedup vs baseline: 2.8136x; 2.8079x over previous
import functools

import jax
import jax.numpy as jnp
from jax import lax
from jax.experimental import pallas as pl
from jax.experimental.pallas import tpu as pltpu
from jax.experimental.pallas import tpu_sc as plsc

VOCAB = 100000
EMBED = 64
BATCH = 1024

_VT = 2048


def _make_sc_gather(V, D, B):
    info = plsc.get_sparse_core_info()
    NC, NS = info.num_cores, info.num_subcores
    NW = NC * NS
    b_per_w = B // NW
    mesh = plsc.VectorSubcoreMesh(core_axis_name="c", subcore_axis_name="s")

    @functools.partial(
        pl.kernel,
        mesh=mesh,
        compiler_params=pltpu.CompilerParams(use_tc_tiling_on_sc=False),
        out_type=jax.ShapeDtypeStruct((B, D), jnp.float32),
        scratch_types=[
            pltpu.VMEM((b_per_w,), jnp.int32),
            pltpu.VMEM((b_per_w, D), jnp.float32),
            pltpu.SemaphoreType.DMA,
        ],
    )
    def gather(table_hbm, idx_hbm, out_hbm, idx_v, rows_v, sem):
        wid = lax.axis_index("s") * NC + lax.axis_index("c")
        base = wid * b_per_w
        pltpu.sync_copy(idx_hbm.at[pl.ds(base, b_per_w)], idx_v)
        pltpu.async_copy(table_hbm.at[idx_v], rows_v, sem).wait()
        pltpu.sync_copy(rows_v, out_hbm.at[pl.ds(base, b_per_w)])

    return gather


def _matmul_body(w_ref, h_ref, o_ref):
    o_ref[...] = lax.dot_general(
        w_ref[...].astype(jnp.bfloat16), h_ref[...].astype(jnp.bfloat16),
        (((0,), (1,)), ((), ())),
        preferred_element_type=jnp.float32,
    )


def kernel(X, W_emb, WT_w):
    hidden = _make_sc_gather(VOCAB, EMBED, BATCH)(W_emb, X.astype(jnp.int32))
    n_tiles = pl.cdiv(VOCAB, _VT)
    outT = pl.pallas_call(
        _matmul_body,
        grid=(n_tiles,),
        in_specs=[
            pl.BlockSpec((EMBED, _VT), lambda i: (0, i)),
            pl.BlockSpec((BATCH, EMBED), lambda i: (0, 0)),
        ],
        out_specs=pl.BlockSpec((_VT, BATCH), lambda i: (i, 0)),
        out_shape=jax.ShapeDtypeStruct((VOCAB, BATCH), jnp.float32),
        compiler_params=pltpu.CompilerParams(
            dimension_semantics=("parallel",),
        ),
    )(WT_w.T, hidden)
    return outT.T


# speedup vs baseline: 3.0603x; 1.0877x over previous
import functools

import jax
import jax.numpy as jnp
from jax import lax
from jax.experimental import pallas as pl
from jax.experimental.pallas import tpu as pltpu
from jax.experimental.pallas import tpu_sc as plsc

VOCAB = 100000
EMBED = 64
BATCH = 1024

_VT = 2048
_CT = 2048


_NB = (VOCAB + _CT - 1) // _CT


def _pack_body(in_ref, o_ref):
    y = in_ref[...].T
    z2 = jnp.concatenate([y[0:_CT // 2, :], y[_CT // 2:_CT, :]], axis=1)
    o_ref[...] = z2.reshape(_CT // 16, 8, 128)


def _pack_table(w_embT):
    return pl.pallas_call(
        _pack_body,
        grid=(_NB,),
        in_specs=[pl.BlockSpec((EMBED, _CT), lambda i: (0, i))],
        out_specs=pl.BlockSpec((_CT // 16, 8, 128), lambda i: (i, 0, 0)),
        out_shape=jax.ShapeDtypeStruct((_NB * _CT // 16, 8, 128), jnp.float32),
        compiler_params=pltpu.CompilerParams(
            dimension_semantics=("parallel",),
        ),
    )(w_embT)


def _make_sc_gather():
    info = plsc.get_sparse_core_info()
    NC, NS = info.num_cores, info.num_subcores
    NW = NC * NS
    b_per_w = BATCH // NW
    mesh = plsc.VectorSubcoreMesh(core_axis_name="c", subcore_axis_name="s")

    @functools.partial(
        pl.kernel,
        mesh=mesh,
        compiler_params=pltpu.CompilerParams(needs_layout_passes=False),
        out_type=jax.ShapeDtypeStruct((BATCH, EMBED), jnp.float32),
        scratch_types=[
            pltpu.VMEM((b_per_w,), jnp.int32),
            pltpu.VMEM((b_per_w,), jnp.int32),
            pltpu.VMEM((b_per_w,), jnp.int32),
            pltpu.VMEM((b_per_w,), jnp.int32),
            pltpu.VMEM((b_per_w, 8, 128), jnp.float32),
            pltpu.VMEM((b_per_w, EMBED), jnp.float32),
            pltpu.SemaphoreType.DMA,
        ],
    )
    def gather(table_hbm, idx_hbm, out_hbm, x_v, t_v, u_v, a_v, rows_v, out_v, sem):
        wid = lax.axis_index("s") * NC + lax.axis_index("c")
        base = wid * b_per_w
        pltpu.sync_copy(idx_hbm.at[pl.ds(base, b_per_w)], x_v)
        for j in range(b_per_w // 16):
            xx = x_v[pl.ds(16 * j, 16)]
            t_v[pl.ds(16 * j, 16)] = (
                lax.shift_right_logical(xx, 11) * 128
                + lax.shift_right_logical(xx & 1023, 3)
            )
            u_v[pl.ds(16 * j, 16)] = xx & 7
            a_v[pl.ds(16 * j, 16)] = lax.shift_right_logical(xx, 10) & 1
        pltpu.async_copy(table_hbm.at[t_v], rows_v, sem).wait()
        lanes = lax.iota(jnp.int32, 16)

        def extract_row(i, carry):
            i16 = jnp.broadcast_to(i, (16,)).astype(jnp.int32)
            u16 = plsc.load_gather(u_v, [i16])
            a16 = plsc.load_gather(a_v, [i16])
            for s in range(EMBED // 16):
                l16 = a16 * 64 + (16 * s) + lanes
                g = plsc.load_gather(rows_v, [i16, u16, l16])
                plsc.store_scatter(out_v, [i16, (16 * s) + lanes], g)
            return carry

        lax.fori_loop(0, b_per_w, extract_row, 0)
        pltpu.sync_copy(out_v, out_hbm.at[pl.ds(base, b_per_w)])

    return gather


def _matmul_body(w_ref, h_ref, o_ref):
    o_ref[...] = lax.dot_general(
        w_ref[...].astype(jnp.bfloat16), h_ref[...].astype(jnp.bfloat16),
        (((0,), (1,)), ((), ())),
        preferred_element_type=jnp.float32,
    )


def kernel(X, W_emb, WT_w):
    packed = _pack_table(W_emb.T)
    hidden = _make_sc_gather()(packed, X.astype(jnp.int32))
    n_tiles = pl.cdiv(VOCAB, _VT)
    outT = pl.pallas_call(
        _matmul_body,
        grid=(n_tiles,),
        in_specs=[
            pl.BlockSpec((EMBED, _VT), lambda i: (0, i)),
            pl.BlockSpec((BATCH, EMBED), lambda i: (0, 0)),
        ],
        out_specs=pl.BlockSpec((_VT, BATCH), lambda i: (i, 0)),
        out_shape=jax.ShapeDtypeStruct((VOCAB, BATCH), jnp.float32),
        compiler_params=pltpu.CompilerParams(
            dimension_semantics=("parallel",),
        ),
    )(WT_w.T, hidden)
    return outT.T


# speedup vs baseline: 3.0918x; 1.0103x over previous
import functools

import jax
import jax.numpy as jnp
from jax import lax
from jax.experimental import pallas as pl
from jax.experimental.pallas import tpu as pltpu
from jax.experimental.pallas import tpu_sc as plsc

VOCAB = 100000
EMBED = 64
BATCH = 1024

_VT = 2048
_CT = 2048


_NB = (VOCAB + _CT - 1) // _CT


def _pack_body(in_ref, o_ref):
    xb = in_ref[...].astype(jnp.bfloat16)
    r = lax.broadcasted_iota(jnp.int32, (EMBED, EMBED), 0)
    c = lax.broadcasted_iota(jnp.int32, (EMBED, EMBED), 1)
    eye = (r == c).astype(jnp.bfloat16)
    y = lax.dot_general(
        xb, eye, (((0,), (0,)), ((), ())),
        preferred_element_type=jnp.float32,
    )
    z2 = jnp.concatenate([y[0:_CT // 2, :], y[_CT // 2:_CT, :]], axis=1)
    o_ref[...] = z2.reshape(_CT // 16, 8, 128)


def _pack_table(w_embT):
    return pl.pallas_call(
        _pack_body,
        grid=(_NB,),
        in_specs=[pl.BlockSpec((EMBED, _CT), lambda i: (0, i))],
        out_specs=pl.BlockSpec((_CT // 16, 8, 128), lambda i: (i, 0, 0)),
        out_shape=jax.ShapeDtypeStruct((_NB * _CT // 16, 8, 128), jnp.float32),
        compiler_params=pltpu.CompilerParams(
            dimension_semantics=("parallel",),
        ),
    )(w_embT)


def _make_sc_gather():
    info = plsc.get_sparse_core_info()
    NC, NS = info.num_cores, info.num_subcores
    NW = NC * NS
    b_per_w = BATCH // NW
    mesh = plsc.VectorSubcoreMesh(core_axis_name="c", subcore_axis_name="s")

    @functools.partial(
        pl.kernel,
        mesh=mesh,
        compiler_params=pltpu.CompilerParams(needs_layout_passes=False),
        out_type=jax.ShapeDtypeStruct((BATCH, EMBED), jnp.float32),
        scratch_types=[
            pltpu.VMEM((b_per_w,), jnp.int32),
            pltpu.VMEM((b_per_w,), jnp.int32),
            pltpu.VMEM((b_per_w,), jnp.int32),
            pltpu.VMEM((b_per_w,), jnp.int32),
            pltpu.VMEM((b_per_w, 8, 128), jnp.float32),
            pltpu.VMEM((b_per_w, EMBED), jnp.float32),
            pltpu.SemaphoreType.DMA,
        ],
    )
    def gather(table_hbm, idx_hbm, out_hbm, x_v, t_v, u_v, a_v, rows_v, out_v, sem):
        wid = lax.axis_index("s") * NC + lax.axis_index("c")
        base = wid * b_per_w
        pltpu.sync_copy(idx_hbm.at[pl.ds(base, b_per_w)], x_v)
        for j in range(b_per_w // 16):
            xx = x_v[pl.ds(16 * j, 16)]
            t_v[pl.ds(16 * j, 16)] = (
                lax.shift_right_logical(xx, 11) * 128
                + lax.shift_right_logical(xx & 1023, 3)
            )
            u_v[pl.ds(16 * j, 16)] = xx & 7
            a_v[pl.ds(16 * j, 16)] = lax.shift_right_logical(xx, 10) & 1
        pltpu.async_copy(table_hbm.at[t_v], rows_v, sem).wait()
        lanes = lax.iota(jnp.int32, 16)

        def extract_row(i, carry):
            i16 = jnp.broadcast_to(i, (16,)).astype(jnp.int32)
            u16 = plsc.load_gather(u_v, [i16])
            a16 = plsc.load_gather(a_v, [i16])
            for s in range(EMBED // 16):
                l16 = a16 * 64 + (16 * s) + lanes
                g = plsc.load_gather(rows_v, [i16, u16, l16])
                plsc.store_scatter(out_v, [i16, (16 * s) + lanes], g)
            return carry

        lax.fori_loop(0, b_per_w, extract_row, 0)
        pltpu.sync_copy(out_v, out_hbm.at[pl.ds(base, b_per_w)])

    return gather


def _matmul_body(w_ref, h_ref, o_ref):
    o_ref[...] = lax.dot_general(
        w_ref[...].astype(jnp.bfloat16), h_ref[...].astype(jnp.bfloat16),
        (((0,), (1,)), ((), ())),
        preferred_element_type=jnp.float32,
    )


def kernel(X, W_emb, WT_w):
    packed = _pack_table(W_emb.T)
    hidden = _make_sc_gather()(packed, X.astype(jnp.int32))
    n_tiles = pl.cdiv(VOCAB, _VT)
    outT = pl.pallas_call(
        _matmul_body,
        grid=(n_tiles,),
        in_specs=[
            pl.BlockSpec((EMBED, _VT), lambda i: (0, i)),
            pl.BlockSpec((BATCH, EMBED), lambda i: (0, 0)),
        ],
        out_specs=pl.BlockSpec((_VT, BATCH), lambda i: (i, 0)),
        out_shape=jax.ShapeDtypeStruct((VOCAB, BATCH), jnp.float32),
        compiler_params=pltpu.CompilerParams(
            dimension_semantics=("parallel",),
        ),
    )(WT_w.T, hidden)
    return outT.T


# speedup vs baseline: 3.4077x; 1.1022x over previous
import functools

import jax
import jax.numpy as jnp
from jax import lax
from jax.experimental import pallas as pl
from jax.experimental.pallas import tpu as pltpu
from jax.experimental.pallas import tpu_sc as plsc

VOCAB = 100000
EMBED = 64
BATCH = 1024

_VT = 2048
_CT = 8192
_CTLOG = 13


_NB = (VOCAB + _CT - 1) // _CT


def _pack_body(in_ref, o_ref):
    xb = in_ref[...].astype(jnp.bfloat16)
    r = lax.broadcasted_iota(jnp.int32, (EMBED, EMBED), 0)
    c = lax.broadcasted_iota(jnp.int32, (EMBED, EMBED), 1)
    eye = (r == c).astype(jnp.bfloat16)
    y = lax.dot_general(
        xb, eye, (((0,), (0,)), ((), ())),
        preferred_element_type=jnp.float32,
    )
    z2 = jnp.concatenate([y[0:_CT // 2, :], y[_CT // 2:_CT, :]], axis=1)
    o_ref[...] = z2.reshape(_CT // 16, 8, 128)


def _pack_table(w_embT):
    return pl.pallas_call(
        _pack_body,
        grid=(_NB,),
        in_specs=[pl.BlockSpec((EMBED, _CT), lambda i: (0, i))],
        out_specs=pl.BlockSpec((_CT // 16, 8, 128), lambda i: (i, 0, 0)),
        out_shape=jax.ShapeDtypeStruct((_NB * _CT // 16, 8, 128), jnp.float32),
        compiler_params=pltpu.CompilerParams(
            dimension_semantics=("parallel",),
        ),
    )(w_embT)


def _make_sc_gather():
    info = plsc.get_sparse_core_info()
    NC, NS = info.num_cores, info.num_subcores
    NW = NC * NS
    b_per_w = BATCH // NW
    mesh = plsc.VectorSubcoreMesh(core_axis_name="c", subcore_axis_name="s")

    @functools.partial(
        pl.kernel,
        mesh=mesh,
        compiler_params=pltpu.CompilerParams(needs_layout_passes=False),
        out_type=jax.ShapeDtypeStruct((BATCH, EMBED), jnp.float32),
        scratch_types=[
            pltpu.VMEM((b_per_w,), jnp.int32),
            pltpu.VMEM((b_per_w,), jnp.int32),
            pltpu.VMEM((b_per_w,), jnp.int32),
            pltpu.VMEM((b_per_w,), jnp.int32),
            pltpu.VMEM((b_per_w, 8, 128), jnp.float32),
            pltpu.VMEM((b_per_w, EMBED), jnp.float32),
            pltpu.SemaphoreType.DMA,
        ],
    )
    def gather(table_hbm, idx_hbm, out_hbm, x_v, t_v, u_v, a_v, rows_v, out_v, sem):
        wid = lax.axis_index("s") * NC + lax.axis_index("c")
        base = wid * b_per_w
        pltpu.sync_copy(idx_hbm.at[pl.ds(base, b_per_w)], x_v)
        for j in range(b_per_w // 16):
            xx = x_v[pl.ds(16 * j, 16)]
            t_v[pl.ds(16 * j, 16)] = (
                lax.shift_right_logical(xx, _CTLOG) * (_CT // 16)
                + lax.shift_right_logical(xx & (_CT // 2 - 1), 3)
            )
            u_v[pl.ds(16 * j, 16)] = xx & 7
            a_v[pl.ds(16 * j, 16)] = lax.shift_right_logical(xx, _CTLOG - 1) & 1
        pltpu.async_copy(table_hbm.at[t_v], rows_v, sem).wait()
        lanes = lax.iota(jnp.int32, 16)

        def extract_row(i, carry):
            i16 = jnp.broadcast_to(i, (16,)).astype(jnp.int32)
            u16 = plsc.load_gather(u_v, [i16])
            a16 = plsc.load_gather(a_v, [i16])
            for s in range(EMBED // 16):
                l16 = a16 * 64 + (16 * s) + lanes
                g = plsc.load_gather(rows_v, [i16, u16, l16])
                plsc.store_scatter(out_v, [i16, (16 * s) + lanes], g)
            return carry

        lax.fori_loop(0, b_per_w, extract_row, 0)
        pltpu.sync_copy(out_v, out_hbm.at[pl.ds(base, b_per_w)])

    return gather


def _matmul_body(w_ref, h_ref, o_ref):
    o_ref[...] = lax.dot_general(
        w_ref[...].astype(jnp.bfloat16), h_ref[...].astype(jnp.bfloat16),
        (((0,), (1,)), ((), ())),
        preferred_element_type=jnp.float32,
    )


def kernel(X, W_emb, WT_w):
    packed = _pack_table(W_emb.T)
    hidden = _make_sc_gather()(packed, X.astype(jnp.int32))
    n_tiles = pl.cdiv(VOCAB, _VT)
    outT = pl.pallas_call(
        _matmul_body,
        grid=(n_tiles,),
        in_specs=[
            pl.BlockSpec((EMBED, _VT), lambda i: (0, i)),
            pl.BlockSpec((BATCH, EMBED), lambda i: (0, 0)),
        ],
        out_specs=pl.BlockSpec((_VT, BATCH), lambda i: (i, 0)),
        out_shape=jax.ShapeDtypeStruct((VOCAB, BATCH), jnp.float32),
        compiler_params=pltpu.CompilerParams(
            dimension_semantics=("parallel",),
        ),
    )(WT_w.T, hidden)
    return outT.T


# speedup vs baseline: 3.4805x; 1.0214x over previous
import functools

import jax
import jax.numpy as jnp
from jax import lax
from jax.experimental import pallas as pl
from jax.experimental.pallas import tpu as pltpu
from jax.experimental.pallas import tpu_sc as plsc

VOCAB = 100000
EMBED = 64
BATCH = 1024

_VT = 4096
_CT = 16384
_CTLOG = 14


_NB = (VOCAB + _CT - 1) // _CT


def _pack_body(in_ref, o_ref):
    xb = in_ref[...].astype(jnp.bfloat16)
    r = lax.broadcasted_iota(jnp.int32, (EMBED, EMBED), 0)
    c = lax.broadcasted_iota(jnp.int32, (EMBED, EMBED), 1)
    eye = (r == c).astype(jnp.bfloat16)
    y = lax.dot_general(
        xb, eye, (((0,), (0,)), ((), ())),
        preferred_element_type=jnp.float32,
    )
    z2 = jnp.concatenate([y[0:_CT // 2, :], y[_CT // 2:_CT, :]], axis=1)
    o_ref[...] = z2.reshape(_CT // 16, 8, 128)


def _pack_table(w_embT):
    return pl.pallas_call(
        _pack_body,
        grid=(_NB,),
        in_specs=[pl.BlockSpec((EMBED, _CT), lambda i: (0, i))],
        out_specs=pl.BlockSpec((_CT // 16, 8, 128), lambda i: (i, 0, 0)),
        out_shape=jax.ShapeDtypeStruct((_NB * _CT // 16, 8, 128), jnp.float32),
        compiler_params=pltpu.CompilerParams(
            dimension_semantics=("parallel",),
        ),
    )(w_embT)


def _make_sc_gather():
    info = plsc.get_sparse_core_info()
    NC, NS = info.num_cores, info.num_subcores
    NW = NC * NS
    b_per_w = BATCH // NW
    mesh = plsc.VectorSubcoreMesh(core_axis_name="c", subcore_axis_name="s")

    @functools.partial(
        pl.kernel,
        mesh=mesh,
        compiler_params=pltpu.CompilerParams(needs_layout_passes=False),
        out_type=jax.ShapeDtypeStruct((BATCH, EMBED), jnp.float32),
        scratch_types=[
            pltpu.VMEM((b_per_w,), jnp.int32),
            pltpu.VMEM((b_per_w,), jnp.int32),
            pltpu.VMEM((b_per_w,), jnp.int32),
            pltpu.VMEM((b_per_w,), jnp.int32),
            pltpu.VMEM((b_per_w, 8, 128), jnp.float32),
            pltpu.VMEM((b_per_w, EMBED), jnp.float32),
            pltpu.SemaphoreType.DMA,
        ],
    )
    def gather(table_hbm, idx_hbm, out_hbm, x_v, t_v, u_v, a_v, rows_v, out_v, sem):
        wid = lax.axis_index("s") * NC + lax.axis_index("c")
        base = wid * b_per_w
        pltpu.sync_copy(idx_hbm.at[pl.ds(base, b_per_w)], x_v)
        for j in range(b_per_w // 16):
            xx = x_v[pl.ds(16 * j, 16)]
            t_v[pl.ds(16 * j, 16)] = (
                lax.shift_right_logical(xx, _CTLOG) * (_CT // 16)
                + lax.shift_right_logical(xx & (_CT // 2 - 1), 3)
            )
            u_v[pl.ds(16 * j, 16)] = xx & 7
            a_v[pl.ds(16 * j, 16)] = lax.shift_right_logical(xx, _CTLOG - 1) & 1
        pltpu.async_copy(table_hbm.at[t_v], rows_v, sem).wait()
        lanes = lax.iota(jnp.int32, 16)

        def extract_row(i, carry):
            i16 = jnp.broadcast_to(i, (16,)).astype(jnp.int32)
            u16 = plsc.load_gather(u_v, [i16])
            a16 = plsc.load_gather(a_v, [i16])
            for s in range(EMBED // 16):
                l16 = a16 * 64 + (16 * s) + lanes
                g = plsc.load_gather(rows_v, [i16, u16, l16])
                plsc.store_scatter(out_v, [i16, (16 * s) + lanes], g)
            return carry

        lax.fori_loop(0, b_per_w, extract_row, 0)
        pltpu.sync_copy(out_v, out_hbm.at[pl.ds(base, b_per_w)])

    return gather


def _matmul_body(w_ref, h_ref, o_ref):
    o_ref[...] = lax.dot_general(
        w_ref[...].astype(jnp.bfloat16), h_ref[...].astype(jnp.bfloat16),
        (((0,), (1,)), ((), ())),
        preferred_element_type=jnp.float32,
    )


def kernel(X, W_emb, WT_w):
    packed = _pack_table(W_emb.T)
    hidden = _make_sc_gather()(packed, X.astype(jnp.int32))
    n_tiles = pl.cdiv(VOCAB, _VT)
    outT = pl.pallas_call(
        _matmul_body,
        grid=(n_tiles,),
        in_specs=[
            pl.BlockSpec((EMBED, _VT), lambda i: (0, i)),
            pl.BlockSpec((BATCH, EMBED), lambda i: (0, 0)),
        ],
        out_specs=pl.BlockSpec((_VT, BATCH), lambda i: (i, 0)),
        out_shape=jax.ShapeDtypeStruct((VOCAB, BATCH), jnp.float32),
        compiler_params=pltpu.CompilerParams(
            dimension_semantics=("parallel",),
        ),
    )(WT_w.T, hidden)
    return outT.T
